# baseline (speedup 1.0000x reference)
import functools

import jax
import jax.numpy as jnp
from jax import lax
from jax.experimental import pallas as pl
from jax.experimental.pallas import tpu as pltpu
from jax.experimental.pallas import tpu_sc as plsc

NUM_EMB = 1000000
D = 32
BATCH = 4096
HIST = 50
TOTAL = BATCH * HIST

NC = 2
NS = 16
NW = NC * NS

PER_W = TOTAL // NW
CHUNK = 128
NCHUNK = PER_W // CHUNK
K = 10
NGROUP = NCHUNK // K
GROUP_ROWS = K * CHUNK

NCOLS = NUM_EMB // 128
TAILC = NCOLS
TAILN = NUM_EMB - NCOLS * 128
ITERS = NCOLS // NW + 1


def _mesh():
    return plsc.VectorSubcoreMesh(
        core_axis_name="c", subcore_axis_name="s", num_cores=NC, num_subcores=NS
    )


@functools.cache
def _build_retile():
    @functools.partial(
        pl.kernel,
        mesh=_mesh(),
        compiler_params=pltpu.CompilerParams(
            use_tc_tiling_on_sc=True, needs_layout_passes=False
        ),
        out_type=jax.ShapeDtypeStruct((NUM_EMB // 32, 8, 128), jnp.float32),
        scratch_types=[
            pltpu.VMEM((2, 4, 8, 128), jnp.float32),
            pltpu.VMEM((2, 4, 8, 128), jnp.float32),
            pltpu.SemaphoreType.DMA,
            pltpu.SemaphoreType.DMA,
            pltpu.SemaphoreType.DMA,
            pltpu.SemaphoreType.DMA,
        ],
    )
    def _retile(w3_hbm, out_hbm, ib, ob, isem0, isem1, osem0, osem1):
        wid = lax.axis_index("s") * NC + lax.axis_index("c")
        isems = (isem0, isem1)
        osems = (osem0, osem1)

        iota = lax.iota(jnp.int32, 16)
        kdiv4 = iota >> 2
        kmod4 = iota & 3
        i2base = kmod4 * 32

        def col_of(i):
            return wid + i * NW

        def fire_in(c, slot):
            for d4 in range(4):
                pltpu.async_copy(
                    w3_hbm.at[pl.ds(d4, 1), :, pl.ds(c * 128, 128)],
                    ib.at[slot, pl.ds(d4, 1)],
                    isems[slot],
                )

        def drain_in(slot, nbytes_shape):
            pltpu.make_async_copy(
                w3_hbm.at[pl.ds(0, nbytes_shape[0]), :, pl.ds(0, nbytes_shape[1])],
                ib.at[slot, pl.ds(0, nbytes_shape[0]), :, pl.ds(0, nbytes_shape[1])],
                isems[slot],
            ).wait()

        def transpose(slot, ngroups):
            for d4 in range(4):
                for s in range(8):
                    i2 = i2base + (8 * d4 + s)
                    for lg in range(ngroups):
                        v = ib[slot, d4, s, pl.ds(16 * lg, 16)]
                        rp = kdiv4 + 4 * lg
                        plsc.store_scatter(
                            ob.at[slot], [rp >> 3, rp & 7, i2], v
                        )

        def fire_out(c, slot):
            pltpu.async_copy(
                ob.at[slot],
                out_hbm.at[pl.ds(c * 4, 4)],
                osems[slot],
            )

        def drain_out(slot, nrows):
            pltpu.make_async_copy(
                ob.at[slot, pl.ds(0, nrows)],
                out_hbm.at[pl.ds(0, nrows)],
                osems[slot],
            ).wait()

        def body(i, carry):
            c = col_of(i)

            def full():
                for d4 in range(4):
                    pltpu.sync_copy(
                        w3_hbm.at[pl.ds(d4, 1), :, pl.ds(c * 128, 128)],
                        ib.at[0, pl.ds(d4, 1)],
                    )
                transpose(0, 8)
                pltpu.sync_copy(ob.at[0], out_hbm.at[pl.ds(c * 4, 4)])

            pl.when(c < NCOLS)(full)
            return carry

        lax.fori_loop(0, ITERS, body, 0)

    return _retile


@functools.cache
def _build_gather():
    @functools.partial(
        pl.kernel,
        mesh=_mesh(),
        compiler_params=pltpu.CompilerParams(
            use_tc_tiling_on_sc=False, needs_layout_passes=False
        ),
        out_type=jax.ShapeDtypeStruct((TOTAL, D), jnp.float32),
        scratch_types=[
            pltpu.VMEM((NCHUNK, CHUNK), jnp.int32),
            pltpu.VMEM((2, GROUP_ROWS, D), jnp.float32),
            pltpu.VMEM((TAILN * D,), jnp.float32),
            pltpu.SemaphoreType.DMA,
            pltpu.SemaphoreType.DMA,
            pltpu.SemaphoreType.DMA,
            pltpu.SemaphoreType.DMA,
        ],
    )
    def _emb_lookup(
        idx_hbm, w_hbm, tail_hbm, out_hbm,
        idx_v, rows_v, tail_v, gsem0, gsem1, wsem0, wsem1,
    ):
        wid = lax.axis_index("s") * NC + lax.axis_index("c")
        base = wid * PER_W
        gsems = (gsem0, gsem1)
        wsems = (wsem0, wsem1)
        tstart = NCOLS * 128

        pltpu.sync_copy(idx_hbm.at[wid], idx_v)
        pltpu.sync_copy(tail_hbm, tail_v)
        iota = lax.iota(jnp.int32, 16)

        def fire_gathers(g, slot):
            for j in range(K):
                pltpu.async_copy(
                    w_hbm.at[idx_v.at[g * K + j]],
                    rows_v.at[slot, pl.ds(j * CHUNK, CHUNK)],
                    gsems[slot],
                )

        def drain_gathers(slot):
            pltpu.make_async_copy(
                w_hbm.at[pl.ds(0, GROUP_ROWS)], rows_v.at[slot], gsems[slot]
            ).wait()

        def fixup(g, slot):
            def gbody(q, carry):
                j = q // 8
                grp = q % 8
                r = idx_v[g * K + j, pl.ds(16 * grp, 16)]
                m = r >= tstart
                anyhit = jnp.max(m.astype(jnp.int32))

                def patch():
                    rt = jnp.maximum(r - tstart, 0) * D
                    row = iota + (j * CHUNK + 16 * grp)

                    def kbody(k, c2):
                        v = plsc.load_gather(tail_v, [rt + k], mask=m)
                        plsc.store_scatter(
                            rows_v.at[slot],
                            [row, jnp.zeros((16,), jnp.int32) + k],
                            v,
                            mask=m,
                        )
                        return c2

                    lax.fori_loop(0, D, kbody, 0)

                pl.when(anyhit > 0)(patch)
                return carry

            lax.fori_loop(0, K * 8, gbody, 0)

        def fire_write(g, slot):
            pltpu.async_copy(
                rows_v.at[slot],
                out_hbm.at[pl.ds(base + g * GROUP_ROWS, GROUP_ROWS)],
                wsems[slot],
            )

        def drain_write(slot):
            pltpu.make_async_copy(
                rows_v.at[slot],
                out_hbm.at[pl.ds(base, GROUP_ROWS)],
                wsems[slot],
            ).wait()

        def per_slot(g, fn):
            pl.when(g % 2 == 0)(lambda: fn(0))
            pl.when(g % 2 == 1)(lambda: fn(1))

        fire_gathers(0, 0)

        def group_body(g, carry):
            pl.when(g >= 2)(lambda: per_slot(g, drain_write))
            per_slot(g, lambda s: fire_gathers(g, s))
            per_slot(g - 1, drain_gathers)
            per_slot(g - 1, lambda s: fixup(g - 1, s))
            per_slot(g - 1, lambda s: fire_write(g - 1, s))
            return carry

        lax.fori_loop(1, NGROUP, group_body, 0)

        last = NGROUP - 1
        per_slot(last, drain_gathers)
        per_slot(last, lambda s: fixup(last, s))
        per_slot(last, lambda s: fire_write(last, s))
        per_slot(last - 1, drain_write)
        per_slot(last, drain_write)

    return _emb_lookup


def kernel(input, weight):
    w3 = weight.T.reshape(4, 8, NUM_EMB)
    table = _build_retile()(w3)
    w_lin = table.reshape(NUM_EMB, D)
    tail = weight[NCOLS * 128 :].reshape(TAILN * D)
    idx = input.astype(jnp.int32).reshape(NW, NCHUNK, CHUNK)
    out = _build_gather()(idx, w_lin, tail)
    return out.reshape(BATCH, HIST, D)


# speedup vs baseline: 1.5836x; 1.5836x over previous
import functools

import jax
import jax.numpy as jnp
from jax import lax
from jax.experimental import pallas as pl
from jax.experimental.pallas import tpu as pltpu
from jax.experimental.pallas import tpu_sc as plsc

NUM_EMB = 1000000
D = 32
BATCH = 4096
HIST = 50
TOTAL = BATCH * HIST

NC = 2
NS = 16
NW = NC * NS

PER_W = TOTAL // NW
CHUNK = 128
NCHUNK = PER_W // CHUNK
K = 10
NGROUP = NCHUNK // K
GROUP_ROWS = K * CHUNK

NCOLS = NUM_EMB // 128
TAILC = NCOLS
TAILN = NUM_EMB - NCOLS * 128
ITERS = NCOLS // NW + 1


def _mesh():
    return plsc.VectorSubcoreMesh(
        core_axis_name="c", subcore_axis_name="s", num_cores=NC, num_subcores=NS
    )


@functools.cache
def _build_retile():
    @functools.partial(
        pl.kernel,
        mesh=_mesh(),
        compiler_params=pltpu.CompilerParams(
            use_tc_tiling_on_sc=True, needs_layout_passes=False
        ),
        out_type=jax.ShapeDtypeStruct((NUM_EMB // 32, 8, 128), jnp.float32),
        scratch_types=[
            pltpu.VMEM((2, 4, 8, 128), jnp.float32),
            pltpu.VMEM((2, 4, 8, 128), jnp.float32),
            pltpu.SemaphoreType.DMA,
            pltpu.SemaphoreType.DMA,
            pltpu.SemaphoreType.DMA,
            pltpu.SemaphoreType.DMA,
        ],
    )
    def _retile(w3_hbm, out_hbm, ib, ob, isem0, isem1, osem0, osem1):
        wid = lax.axis_index("s") * NC + lax.axis_index("c")
        isems = (isem0, isem1)
        osems = (osem0, osem1)

        iota = lax.iota(jnp.int32, 16)
        kdiv4 = iota >> 2
        kmod4 = iota & 3
        i2base = kmod4 * 32

        def col_of(i):
            return wid + i * NW

        def fire_in(c, slot):
            for d4 in range(4):
                pltpu.async_copy(
                    w3_hbm.at[pl.ds(d4, 1), :, pl.ds(c * 128, 128)],
                    ib.at[slot, pl.ds(d4, 1)],
                    isems[slot],
                )

        def drain_in(slot, nbytes_shape):
            pltpu.make_async_copy(
                w3_hbm.at[pl.ds(0, nbytes_shape[0]), :, pl.ds(0, nbytes_shape[1])],
                ib.at[slot, pl.ds(0, nbytes_shape[0]), :, pl.ds(0, nbytes_shape[1])],
                isems[slot],
            ).wait()

        def transpose(slot, ngroups):
            for d4 in range(4):
                for s in range(8):
                    i2 = i2base + (8 * d4 + s)
                    for lg in range(ngroups):
                        v = ib[slot, d4, s, pl.ds(16 * lg, 16)]
                        rp = kdiv4 + 4 * lg
                        plsc.store_scatter(
                            ob.at[slot], [rp >> 3, rp & 7, i2], v
                        )

        def fire_out(c, slot):
            pltpu.async_copy(
                ob.at[slot],
                out_hbm.at[pl.ds(c * 4, 4)],
                osems[slot],
            )

        def drain_out(slot, nrows):
            pltpu.make_async_copy(
                ob.at[slot, pl.ds(0, nrows)],
                out_hbm.at[pl.ds(0, nrows)],
                osems[slot],
            ).wait()

        NPIPE = 244

        def consume(i, slot):
            drain_in(slot, (4, 128))
            transpose(slot, 8)
            fire_out(col_of(i), slot)

        def per_slot(i, fn):
            pl.when(i % 2 == 0)(lambda: fn(0))
            pl.when(i % 2 == 1)(lambda: fn(1))

        fire_in(col_of(0), 0)

        def body(i, carry):
            def prep(slot):
                pl.when(i >= 2)(lambda: drain_out(slot, 4))
                fire_in(col_of(i), slot)

            per_slot(i, prep)
            per_slot(i - 1, lambda s: consume(i - 1, s))
            return carry

        lax.fori_loop(1, NPIPE, body, 0)

        last = NPIPE - 1
        per_slot(last, lambda s: consume(last, s))
        per_slot(last - 1, lambda s: drain_out(s, 4))
        per_slot(last, lambda s: drain_out(s, 4))

        def leftover():
            c = NW * NPIPE + wid
            for d4 in range(4):
                pltpu.sync_copy(
                    w3_hbm.at[pl.ds(d4, 1), :, pl.ds(c * 128, 128)],
                    ib.at[0, pl.ds(d4, 1)],
                )
            transpose(0, 8)
            pltpu.sync_copy(ob.at[0], out_hbm.at[pl.ds(c * 4, 4)])

        pl.when(wid < NCOLS - NW * NPIPE)(leftover)

    return _retile


@functools.cache
def _build_gather():
    @functools.partial(
        pl.kernel,
        mesh=_mesh(),
        compiler_params=pltpu.CompilerParams(
            use_tc_tiling_on_sc=False, needs_layout_passes=False
        ),
        out_type=jax.ShapeDtypeStruct((TOTAL, D), jnp.float32),
        scratch_types=[
            pltpu.VMEM((NCHUNK, CHUNK), jnp.int32),
            pltpu.VMEM((2, GROUP_ROWS, D), jnp.float32),
            pltpu.VMEM((TAILN * D,), jnp.float32),
            pltpu.SemaphoreType.DMA,
            pltpu.SemaphoreType.DMA,
            pltpu.SemaphoreType.DMA,
            pltpu.SemaphoreType.DMA,
        ],
    )
    def _emb_lookup(
        idx_hbm, w_hbm, tail_hbm, out_hbm,
        idx_v, rows_v, tail_v, gsem0, gsem1, wsem0, wsem1,
    ):
        wid = lax.axis_index("s") * NC + lax.axis_index("c")
        base = wid * PER_W
        gsems = (gsem0, gsem1)
        wsems = (wsem0, wsem1)
        tstart = NCOLS * 128

        pltpu.sync_copy(idx_hbm.at[wid], idx_v)
        pltpu.sync_copy(tail_hbm, tail_v)
        iota = lax.iota(jnp.int32, 16)

        def fire_gathers(g, slot):
            for j in range(K):
                pltpu.async_copy(
                    w_hbm.at[idx_v.at[g * K + j]],
                    rows_v.at[slot, pl.ds(j * CHUNK, CHUNK)],
                    gsems[slot],
                )

        def drain_gathers(slot):
            pltpu.make_async_copy(
                w_hbm.at[pl.ds(0, GROUP_ROWS)], rows_v.at[slot], gsems[slot]
            ).wait()

        def fixup(g, slot):
            def gbody(q, carry):
                j = q // 8
                grp = q % 8
                r = idx_v[g * K + j, pl.ds(16 * grp, 16)]
                m = r >= tstart
                anyhit = jnp.max(m.astype(jnp.int32))

                def patch():
                    rt = jnp.maximum(r - tstart, 0) * D
                    row = iota + (j * CHUNK + 16 * grp)

                    def kbody(k, c2):
                        v = plsc.load_gather(tail_v, [rt + k], mask=m)
                        plsc.store_scatter(
                            rows_v.at[slot],
                            [row, jnp.zeros((16,), jnp.int32) + k],
                            v,
                            mask=m,
                        )
                        return c2

                    lax.fori_loop(0, D, kbody, 0)

                pl.when(anyhit > 0)(patch)
                return carry

            lax.fori_loop(0, K * 8, gbody, 0)

        def fire_write(g, slot):
            pltpu.async_copy(
                rows_v.at[slot],
                out_hbm.at[pl.ds(base + g * GROUP_ROWS, GROUP_ROWS)],
                wsems[slot],
            )

        def drain_write(slot):
            pltpu.make_async_copy(
                rows_v.at[slot],
                out_hbm.at[pl.ds(base, GROUP_ROWS)],
                wsems[slot],
            ).wait()

        def per_slot(g, fn):
            pl.when(g % 2 == 0)(lambda: fn(0))
            pl.when(g % 2 == 1)(lambda: fn(1))

        fire_gathers(0, 0)

        def group_body(g, carry):
            pl.when(g >= 2)(lambda: per_slot(g, drain_write))
            per_slot(g, lambda s: fire_gathers(g, s))
            per_slot(g - 1, drain_gathers)
            per_slot(g - 1, lambda s: fixup(g - 1, s))
            per_slot(g - 1, lambda s: fire_write(g - 1, s))
            return carry

        lax.fori_loop(1, NGROUP, group_body, 0)

        last = NGROUP - 1
        per_slot(last, drain_gathers)
        per_slot(last, lambda s: fixup(last, s))
        per_slot(last, lambda s: fire_write(last, s))
        per_slot(last - 1, drain_write)
        per_slot(last, drain_write)

    return _emb_lookup


def kernel(input, weight):
    w3 = weight.T.reshape(4, 8, NUM_EMB)
    table = _build_retile()(w3)
    w_lin = table.reshape(NUM_EMB, D)
    tail = weight[NCOLS * 128 :].reshape(TAILN * D)
    idx = input.astype(jnp.int32).reshape(NW, NCHUNK, CHUNK)
    out = _build_gather()(idx, w_lin, tail)
    return out.reshape(BATCH, HIST, D)


# speedup vs baseline: 1.5973x; 1.0087x over previous
import functools

import jax
import jax.numpy as jnp
from jax import lax
from jax.experimental import pallas as pl
from jax.experimental.pallas import tpu as pltpu
from jax.experimental.pallas import tpu_sc as plsc

NUM_EMB = 1000000
D = 32
BATCH = 4096
HIST = 50
TOTAL = BATCH * HIST

NC = 2
NS = 16
NW = NC * NS

PER_W = TOTAL // NW
CHUNK = 128
NCHUNK = PER_W // CHUNK
K = 10
NGROUP = NCHUNK // K
GROUP_ROWS = K * CHUNK

NCOLS = NUM_EMB // 128
TAILC = NCOLS
TAILN = NUM_EMB - NCOLS * 128
ITERS = NCOLS // NW + 1


def _mesh():
    return plsc.VectorSubcoreMesh(
        core_axis_name="c", subcore_axis_name="s", num_cores=NC, num_subcores=NS
    )


@functools.cache
def _build_retile():
    @functools.partial(
        pl.kernel,
        mesh=_mesh(),
        compiler_params=pltpu.CompilerParams(
            use_tc_tiling_on_sc=True, needs_layout_passes=False
        ),
        out_type=jax.ShapeDtypeStruct((NUM_EMB * D,), jnp.float32),
        scratch_types=[
            pltpu.VMEM((2, 4, 8, 512), jnp.float32),
            pltpu.VMEM((32768,), jnp.float32),
            pltpu.SemaphoreType.DMA,
            pltpu.SemaphoreType.DMA,
            pltpu.SemaphoreType.DMA,
            pltpu.SemaphoreType.DMA,
        ],
    )
    def _retile(w3_hbm, out_hbm, ib, ob, isem0, isem1, osem0, osem1):
        wid = lax.axis_index("s") * NC + lax.axis_index("c")
        isems = (isem0, isem1)
        osems = (osem0, osem1)
        cstart = wid * 244
        NBLK = 61

        iota = lax.iota(jnp.int32, 16)
        kdiv4 = iota >> 2
        kmod4 = iota & 3
        vb = kdiv4 * 128 + kmod4 * 32

        def c0_of(i):
            return cstart + i * 4

        def fire_in(c0, slot):
            for d4 in range(4):
                pltpu.async_copy(
                    w3_hbm.at[pl.ds(d4, 1), :, pl.ds(c0 * 128, 512)],
                    ib.at[slot, pl.ds(d4, 1)],
                    isems[slot],
                )

        def drain_in(slot):
            pltpu.make_async_copy(
                w3_hbm.at[pl.ds(0, 4), :, pl.ds(0, 512)],
                ib.at[slot],
                isems[slot],
            ).wait()

        def transpose(slot):
            for d4 in range(4):
                for s in range(8):
                    for t0 in range(0, 32, 8):
                        vs = [
                            ib[slot, d4, s, pl.ds(16 * (t0 + t), 16)]
                            for t in range(8)
                        ]
                        for t, v in enumerate(vs):
                            tt = t0 + t
                            j, lg = tt // 8, tt % 8
                            off = 4096 * j + 512 * lg + 8 * d4 + s
                            plsc.store_scatter(
                                ob.at[pl.ds(slot * 16384, 16384)],
                                [vb + off], v,
                            )

        def fire_out(c0, slot):
            pltpu.async_copy(
                ob.at[pl.ds(slot * 16384, 16384)],
                out_hbm.at[pl.ds(c0 * 4096, 16384)],
                osems[slot],
            )

        def drain_out(slot):
            pltpu.make_async_copy(
                ob.at[pl.ds(slot * 16384, 16384)],
                out_hbm.at[pl.ds(0, 16384)],
                osems[slot],
            ).wait()

        def consume(i, slot):
            drain_in(slot)
            transpose(slot)
            fire_out(c0_of(i), slot)

        def per_slot(i, fn):
            pl.when(i % 2 == 0)(lambda: fn(0))
            pl.when(i % 2 == 1)(lambda: fn(1))

        fire_in(c0_of(0), 0)

        def body(i, carry):
            def prep(slot):
                pl.when(i >= 2)(lambda: drain_out(slot))
                fire_in(c0_of(i), slot)

            per_slot(i, prep)
            per_slot(i - 1, lambda s: consume(i - 1, s))
            return carry

        lax.fori_loop(1, NBLK, body, 0)

        last = NBLK - 1
        per_slot(last, lambda s: consume(last, s))
        per_slot(last - 1, drain_out)
        per_slot(last, drain_out)

        def leftover():
            c = NW * 244 + wid
            for d4 in range(4):
                pltpu.sync_copy(
                    w3_hbm.at[pl.ds(d4, 1), :, pl.ds(c * 128, 128)],
                    ib.at[0, pl.ds(d4, 1), :, pl.ds(0, 128)],
                )
            for d4 in range(4):
                for s in range(8):
                    for lg in range(8):
                        v = ib[0, d4, s, pl.ds(16 * lg, 16)]
                        off = 512 * lg + 8 * d4 + s
                        plsc.store_scatter(ob.at[pl.ds(0, 4096)], [vb + off], v)
            pltpu.sync_copy(
                ob.at[pl.ds(0, 4096)],
                out_hbm.at[pl.ds(c * 4096, 4096)],
            )

        pl.when(wid < NCOLS - NW * 244)(leftover)

    return _retile


@functools.cache
def _build_gather():
    @functools.partial(
        pl.kernel,
        mesh=_mesh(),
        compiler_params=pltpu.CompilerParams(
            use_tc_tiling_on_sc=False, needs_layout_passes=False
        ),
        out_type=jax.ShapeDtypeStruct((TOTAL, D), jnp.float32),
        scratch_types=[
            pltpu.VMEM((NCHUNK, CHUNK), jnp.int32),
            pltpu.VMEM((2, GROUP_ROWS, D), jnp.float32),
            pltpu.VMEM((TAILN * D,), jnp.float32),
            pltpu.SemaphoreType.DMA,
            pltpu.SemaphoreType.DMA,
            pltpu.SemaphoreType.DMA,
            pltpu.SemaphoreType.DMA,
        ],
    )
    def _emb_lookup(
        idx_hbm, w_hbm, tail_hbm, out_hbm,
        idx_v, rows_v, tail_v, gsem0, gsem1, wsem0, wsem1,
    ):
        wid = lax.axis_index("s") * NC + lax.axis_index("c")
        base = wid * PER_W
        gsems = (gsem0, gsem1)
        wsems = (wsem0, wsem1)
        tstart = NCOLS * 128

        pltpu.sync_copy(idx_hbm.at[wid], idx_v)
        pltpu.sync_copy(tail_hbm, tail_v)
        iota = lax.iota(jnp.int32, 16)

        def fire_gathers(g, slot):
            for j in range(K):
                pltpu.async_copy(
                    w_hbm.at[idx_v.at[g * K + j]],
                    rows_v.at[slot, pl.ds(j * CHUNK, CHUNK)],
                    gsems[slot],
                )

        def drain_gathers(slot):
            pltpu.make_async_copy(
                w_hbm.at[pl.ds(0, GROUP_ROWS)], rows_v.at[slot], gsems[slot]
            ).wait()

        def fixup(g, slot):
            def gbody(q, carry):
                j = q // 8
                grp = q % 8
                r = idx_v[g * K + j, pl.ds(16 * grp, 16)]
                m = r >= tstart
                anyhit = jnp.max(m.astype(jnp.int32))

                def patch():
                    rt = jnp.maximum(r - tstart, 0) * D
                    row = iota + (j * CHUNK + 16 * grp)

                    def kbody(k, c2):
                        v = plsc.load_gather(tail_v, [rt + k], mask=m)
                        plsc.store_scatter(
                            rows_v.at[slot],
                            [row, jnp.zeros((16,), jnp.int32) + k],
                            v,
                            mask=m,
                        )
                        return c2

                    lax.fori_loop(0, D, kbody, 0)

                pl.when(anyhit > 0)(patch)
                return carry

            lax.fori_loop(0, K * 8, gbody, 0)

        def fire_write(g, slot):
            pltpu.async_copy(
                rows_v.at[slot],
                out_hbm.at[pl.ds(base + g * GROUP_ROWS, GROUP_ROWS)],
                wsems[slot],
            )

        def drain_write(slot):
            pltpu.make_async_copy(
                rows_v.at[slot],
                out_hbm.at[pl.ds(base, GROUP_ROWS)],
                wsems[slot],
            ).wait()

        def per_slot(g, fn):
            pl.when(g % 2 == 0)(lambda: fn(0))
            pl.when(g % 2 == 1)(lambda: fn(1))

        fire_gathers(0, 0)

        def group_body(g, carry):
            pl.when(g >= 2)(lambda: per_slot(g, drain_write))
            per_slot(g, lambda s: fire_gathers(g, s))
            per_slot(g - 1, drain_gathers)
            per_slot(g - 1, lambda s: fixup(g - 1, s))
            per_slot(g - 1, lambda s: fire_write(g - 1, s))
            return carry

        lax.fori_loop(1, NGROUP, group_body, 0)

        last = NGROUP - 1
        per_slot(last, drain_gathers)
        per_slot(last, lambda s: fixup(last, s))
        per_slot(last, lambda s: fire_write(last, s))
        per_slot(last - 1, drain_write)
        per_slot(last, drain_write)

    return _emb_lookup


def kernel(input, weight):
    w3 = weight.T.reshape(4, 8, NUM_EMB)
    table = _build_retile()(w3)
    w_lin = table.reshape(NUM_EMB, D)
    tail = weight[NCOLS * 128 :].reshape(TAILN * D)
    idx = input.astype(jnp.int32).reshape(NW, NCHUNK, CHUNK)
    out = _build_gather()(idx, w_lin, tail)
    return out.reshape(BATCH, HIST, D)


# speedup vs baseline: 1.5994x; 1.0013x over previous
import functools

import jax
import jax.numpy as jnp
from jax import lax
from jax.experimental import pallas as pl
from jax.experimental.pallas import tpu as pltpu
from jax.experimental.pallas import tpu_sc as plsc

NUM_EMB = 1000000
D = 32
BATCH = 4096
HIST = 50
TOTAL = BATCH * HIST

NC = 2
NS = 16
NW = NC * NS

PER_W = TOTAL // NW
CHUNK = 128
NCHUNK = PER_W // CHUNK
K = 10
NGROUP = NCHUNK // K
GROUP_ROWS = K * CHUNK

NCOLS = NUM_EMB // 128
TAILC = NCOLS
TAILN = NUM_EMB - NCOLS * 128
ITERS = NCOLS // NW + 1


def _mesh():
    return plsc.VectorSubcoreMesh(
        core_axis_name="c", subcore_axis_name="s", num_cores=NC, num_subcores=NS
    )


@functools.cache
def _build_retile():
    @functools.partial(
        pl.kernel,
        mesh=_mesh(),
        compiler_params=pltpu.CompilerParams(
            use_tc_tiling_on_sc=True, needs_layout_passes=False
        ),
        out_type=jax.ShapeDtypeStruct((NUM_EMB * D,), jnp.float32),
        scratch_types=[
            pltpu.VMEM((2, 4, 8, 512), jnp.float32),
            pltpu.VMEM((32768,), jnp.float32),
            pltpu.SemaphoreType.DMA,
            pltpu.SemaphoreType.DMA,
            pltpu.SemaphoreType.DMA,
            pltpu.SemaphoreType.DMA,
        ],
    )
    def _retile(w3_hbm, out_hbm, ib, ob, isem0, isem1, osem0, osem1):
        wid = lax.axis_index("s") * NC + lax.axis_index("c")
        isems = (isem0, isem1)
        osems = (osem0, osem1)
        cstart = wid * 244
        NBLK = 61

        iota = lax.iota(jnp.int32, 16)
        kdiv4 = iota >> 2
        kmod4 = iota & 3
        vb = kdiv4 * 128 + kmod4 * 32
        vbs = [vb + s for s in range(8)]

        def c0_of(i):
            return cstart + i * 4

        def fire_in(c0, slot):
            for d4 in range(4):
                pltpu.async_copy(
                    w3_hbm.at[pl.ds(d4, 1), :, pl.ds(c0 * 128, 512)],
                    ib.at[slot, pl.ds(d4, 1)],
                    isems[slot],
                )

        def drain_in(slot):
            pltpu.make_async_copy(
                w3_hbm.at[pl.ds(0, 4), :, pl.ds(0, 512)],
                ib.at[slot],
                isems[slot],
            ).wait()

        def transpose(slot):
            for d4 in range(4):
                for s in range(8):
                    for t0 in range(0, 32, 8):
                        vs = [
                            ib[slot, d4, s, pl.ds(16 * (t0 + t), 16)]
                            for t in range(8)
                        ]
                        for t, v in enumerate(vs):
                            tt = t0 + t
                            j, lg = tt // 8, tt % 8
                            off = 4096 * j + 512 * lg + 8 * d4
                            plsc.store_scatter(
                                ob.at[pl.ds(slot * 16384 + off, 488)],
                                [vbs[s]], v,
                            )

        def fire_out(c0, slot):
            pltpu.async_copy(
                ob.at[pl.ds(slot * 16384, 16384)],
                out_hbm.at[pl.ds(c0 * 4096, 16384)],
                osems[slot],
            )

        def drain_out(slot):
            pltpu.make_async_copy(
                ob.at[pl.ds(slot * 16384, 16384)],
                out_hbm.at[pl.ds(0, 16384)],
                osems[slot],
            ).wait()

        def consume(i, slot):
            drain_in(slot)
            transpose(slot)
            fire_out(c0_of(i), slot)

        def per_slot(i, fn):
            pl.when(i % 2 == 0)(lambda: fn(0))
            pl.when(i % 2 == 1)(lambda: fn(1))

        fire_in(c0_of(0), 0)

        def body(i, carry):
            def prep(slot):
                pl.when(i >= 2)(lambda: drain_out(slot))
                fire_in(c0_of(i), slot)

            per_slot(i, prep)
            per_slot(i - 1, lambda s: consume(i - 1, s))
            return carry

        lax.fori_loop(1, NBLK, body, 0)

        last = NBLK - 1
        per_slot(last, lambda s: consume(last, s))
        per_slot(last - 1, drain_out)
        per_slot(last, drain_out)

        def leftover():
            c = NW * 244 + wid
            for d4 in range(4):
                pltpu.sync_copy(
                    w3_hbm.at[pl.ds(d4, 1), :, pl.ds(c * 128, 128)],
                    ib.at[0, pl.ds(d4, 1), :, pl.ds(0, 128)],
                )
            for d4 in range(4):
                for s in range(8):
                    for lg in range(8):
                        v = ib[0, d4, s, pl.ds(16 * lg, 16)]
                        off = 512 * lg + 8 * d4
                        plsc.store_scatter(ob.at[pl.ds(off, 488)], [vbs[s]], v)
            pltpu.sync_copy(
                ob.at[pl.ds(0, 4096)],
                out_hbm.at[pl.ds(c * 4096, 4096)],
            )

        pl.when(wid < NCOLS - NW * 244)(leftover)

    return _retile


@functools.cache
def _build_gather():
    @functools.partial(
        pl.kernel,
        mesh=_mesh(),
        compiler_params=pltpu.CompilerParams(
            use_tc_tiling_on_sc=False, needs_layout_passes=False
        ),
        out_type=jax.ShapeDtypeStruct((TOTAL, D), jnp.float32),
        scratch_types=[
            pltpu.VMEM((NCHUNK, CHUNK), jnp.int32),
            pltpu.VMEM((2, GROUP_ROWS, D), jnp.float32),
            pltpu.VMEM((TAILN * D,), jnp.float32),
            pltpu.SemaphoreType.DMA,
            pltpu.SemaphoreType.DMA,
            pltpu.SemaphoreType.DMA,
            pltpu.SemaphoreType.DMA,
        ],
    )
    def _emb_lookup(
        idx_hbm, w_hbm, tail_hbm, out_hbm,
        idx_v, rows_v, tail_v, gsem0, gsem1, wsem0, wsem1,
    ):
        wid = lax.axis_index("s") * NC + lax.axis_index("c")
        base = wid * PER_W
        gsems = (gsem0, gsem1)
        wsems = (wsem0, wsem1)
        tstart = NCOLS * 128

        pltpu.sync_copy(idx_hbm.at[wid], idx_v)
        pltpu.sync_copy(tail_hbm, tail_v)
        iota = lax.iota(jnp.int32, 16)

        def fire_gathers(g, slot):
            for j in range(K):
                pltpu.async_copy(
                    w_hbm.at[idx_v.at[g * K + j]],
                    rows_v.at[slot, pl.ds(j * CHUNK, CHUNK)],
                    gsems[slot],
                )

        def drain_gathers(slot):
            pltpu.make_async_copy(
                w_hbm.at[pl.ds(0, GROUP_ROWS)], rows_v.at[slot], gsems[slot]
            ).wait()

        def fixup(g, slot):
            def gbody(q, carry):
                j = q // 8
                grp = q % 8
                r = idx_v[g * K + j, pl.ds(16 * grp, 16)]
                m = r >= tstart
                anyhit = jnp.max(m.astype(jnp.int32))

                def patch():
                    rt = jnp.maximum(r - tstart, 0) * D
                    row = iota + (j * CHUNK + 16 * grp)

                    def kbody(k, c2):
                        v = plsc.load_gather(tail_v, [rt + k], mask=m)
                        plsc.store_scatter(
                            rows_v.at[slot],
                            [row, jnp.zeros((16,), jnp.int32) + k],
                            v,
                            mask=m,
                        )
                        return c2

                    lax.fori_loop(0, D, kbody, 0)

                pl.when(anyhit > 0)(patch)
                return carry

            lax.fori_loop(0, K * 8, gbody, 0)

        def fire_write(g, slot):
            pltpu.async_copy(
                rows_v.at[slot],
                out_hbm.at[pl.ds(base + g * GROUP_ROWS, GROUP_ROWS)],
                wsems[slot],
            )

        def drain_write(slot):
            pltpu.make_async_copy(
                rows_v.at[slot],
                out_hbm.at[pl.ds(base, GROUP_ROWS)],
                wsems[slot],
            ).wait()

        def per_slot(g, fn):
            pl.when(g % 2 == 0)(lambda: fn(0))
            pl.when(g % 2 == 1)(lambda: fn(1))

        fire_gathers(0, 0)

        def group_body(g, carry):
            pl.when(g >= 2)(lambda: per_slot(g, drain_write))
            per_slot(g, lambda s: fire_gathers(g, s))
            per_slot(g - 1, drain_gathers)
            per_slot(g - 1, lambda s: fixup(g - 1, s))
            per_slot(g - 1, lambda s: fire_write(g - 1, s))
            return carry

        lax.fori_loop(1, NGROUP, group_body, 0)

        last = NGROUP - 1
        per_slot(last, drain_gathers)
        per_slot(last, lambda s: fixup(last, s))
        per_slot(last, lambda s: fire_write(last, s))
        per_slot(last - 1, drain_write)
        per_slot(last, drain_write)

    return _emb_lookup


def kernel(input, weight):
    w3 = weight.T.reshape(4, 8, NUM_EMB)
    table = _build_retile()(w3)
    w_lin = table.reshape(NUM_EMB, D)
    tail = weight[NCOLS * 128 :].reshape(TAILN * D)
    idx = input.astype(jnp.int32).reshape(NW, NCHUNK, CHUNK)
    out = _build_gather()(idx, w_lin, tail)
    return out.reshape(BATCH, HIST, D)


# speedup vs baseline: 1.9675x; 1.2302x over previous
import functools

import jax
import jax.numpy as jnp
from jax import lax
from jax.experimental import pallas as pl
from jax.experimental.pallas import tpu as pltpu
from jax.experimental.pallas import tpu_sc as plsc

NUM_EMB = 1000000
D = 32
BATCH = 4096
HIST = 50
TOTAL = BATCH * HIST

NC = 2
NS = 16
NW = NC * NS
PER_W = TOTAL // NW
CHUNK = 128
NCHUNK = PER_W // CHUNK
K = 10
NGROUP = NCHUNK // K
GROUP_ROWS = K * CHUNK


@functools.cache
def _build_kernel():
    mesh = plsc.VectorSubcoreMesh(
        core_axis_name="c", subcore_axis_name="s", num_cores=NC, num_subcores=NS
    )

    @functools.partial(
        pl.kernel,
        mesh=mesh,
        compiler_params=pltpu.CompilerParams(use_tc_tiling_on_sc=False),
        out_type=jax.ShapeDtypeStruct((TOTAL, D), jnp.float32),
        scratch_types=[
            pltpu.VMEM((NCHUNK, CHUNK), jnp.int32),
            pltpu.VMEM((2, GROUP_ROWS, D), jnp.float32),
            pltpu.SemaphoreType.DMA,
            pltpu.SemaphoreType.DMA,
            pltpu.SemaphoreType.DMA,
            pltpu.SemaphoreType.DMA,
        ],
    )
    def _emb_lookup(
        idx_hbm, w_hbm, out_hbm, idx_v, rows_v, gsem0, gsem1, wsem0, wsem1
    ):
        wid = lax.axis_index("s") * NC + lax.axis_index("c")
        base = wid * PER_W
        gsems = (gsem0, gsem1)
        wsems = (wsem0, wsem1)

        pltpu.sync_copy(idx_hbm.at[wid], idx_v)

        def fire_gathers(g, slot):
            for j in range(K):
                pltpu.async_copy(
                    w_hbm.at[idx_v.at[g * K + j]],
                    rows_v.at[slot, pl.ds(j * CHUNK, CHUNK)],
                    gsems[slot],
                )

        def drain_gathers(slot):
            pltpu.make_async_copy(
                w_hbm.at[pl.ds(0, GROUP_ROWS)], rows_v.at[slot], gsems[slot]
            ).wait()

        def fire_write(g, slot):
            pltpu.async_copy(
                rows_v.at[slot],
                out_hbm.at[pl.ds(base + g * GROUP_ROWS, GROUP_ROWS)],
                wsems[slot],
            )

        def drain_write(slot):
            pltpu.make_async_copy(
                rows_v.at[slot],
                out_hbm.at[pl.ds(base, GROUP_ROWS)],
                wsems[slot],
            ).wait()

        def per_slot(g, fn):
            pl.when(g % 2 == 0)(lambda: fn(0))
            pl.when(g % 2 == 1)(lambda: fn(1))

        fire_gathers(0, 0)

        def group_body(g, carry):
            pl.when(g >= 2)(lambda: per_slot(g, drain_write))
            per_slot(g, lambda s: fire_gathers(g, s))
            per_slot(g - 1, drain_gathers)
            per_slot(g - 1, lambda s: fire_write(g - 1, s))
            return carry

        lax.fori_loop(1, NGROUP, group_body, 0)

        last = NGROUP - 1
        per_slot(last, drain_gathers)
        per_slot(last, lambda s: fire_write(last, s))
        per_slot(last - 1, drain_write)
        per_slot(last, drain_write)

    return _emb_lookup


def kernel(input, weight):
    idx = input.astype(jnp.int32).reshape(NW, NCHUNK, CHUNK)
    out = _build_kernel()(idx, weight)
    return out.reshape(BATCH, HIST, D)
